# baseline (speedup 1.0000x reference)
import functools

import jax
import jax.numpy as jnp
from jax import lax
from jax.experimental import pallas as pl
from jax.experimental.pallas import tpu as pltpu
from jax.experimental.pallas import tpu_sc as plsc

B, L, D = 16, 512, 512
T = 2048
LANES = 16
NC, NS = 2, 16
NW = NC * NS
CHUNK = 64
NBUF = 3
HALF = T // 2
NCH = HALF // CHUNK
PAD_ROWS = 8
SENTINEL = B * L


def _lr_body(xf, dur, out, tot, dur_v, idx_buf, tot_v,
             b0, b1, b2, g0, g1, g2, o0, o1, o2):
    bufs = (b0, b1, b2)
    gsem = (g0, g1, g2)
    osem = (o0, o1, o2)

    wid = lax.axis_index("c") * NS + lax.axis_index("s")
    b = wid // 2
    h = wid % 2

    pltpu.sync_copy(dur.at[b], dur_v)

    sent = jnp.full((LANES,), SENTINEL, jnp.int32)
    for r in range(T // CHUNK):
        for j in range(CHUNK // LANES):
            idx_buf[r, pl.ds(j * LANES, LANES)] = sent

    iota = lax.iota(jnp.int32, LANES)
    row_base = b * L

    def cs_body(i, carry):
        ch = dur_v[pl.ds(i * LANES, LANES)]
        inc = plsc.cumsum(ch)
        excl = carry + inc - ch
        src = row_base + i * LANES + iota
        for k in range(7):
            pos = excl + k
            m = (ch > k) & (pos < T)
            posc = jnp.minimum(pos, T - 1)
            plsc.store_scatter(idx_buf, [posc >> 6, posc & (CHUNK - 1)],
                               src, mask=m)
        return carry + jnp.sum(ch)

    total = lax.fori_loop(0, L // LANES, cs_body, jnp.int32(0))

    @pl.when(h == 0)
    def _():
        tot_v[...] = jnp.full((LANES,), total, jnp.int32)
        pltpu.sync_copy(tot_v, tot.at[b])

    row0 = h * NCH

    def g_start(c, buf, sem):
        return pltpu.async_copy(xf.at[idx_buf.at[row0 + c]], buf, sem)

    def o_start(c, buf, sem):
        dst = out.at[b, pl.ds(h * HALF + c * CHUNK, CHUNK)]
        return pltpu.async_copy(buf, dst, sem)

    gh = {}
    oh = {}
    for c in range(min(NBUF, NCH)):
        gh[c] = g_start(c, bufs[c % NBUF], gsem[c % NBUF])
    for c in range(NCH):
        i = c % NBUF
        gh[c].wait()
        oh[c] = o_start(c, bufs[i], osem[i])
        n = c + NBUF
        if n < NCH:
            oh[c].wait()
            gh[n] = g_start(n, bufs[i], gsem[i])
    for c in range(max(0, NCH - NBUF), NCH):
        oh[c].wait()


def kernel(x, duration, max_len):
    xf = jnp.concatenate(
        [x.reshape(B * L, D), jnp.zeros((PAD_ROWS, D), x.dtype)], axis=0)
    mesh = plsc.VectorSubcoreMesh(core_axis_name="c", subcore_axis_name="s")
    out, tot = pl.kernel(
        _lr_body,
        out_type=[
            jax.ShapeDtypeStruct((B, T, D), x.dtype),
            jax.ShapeDtypeStruct((B, LANES), jnp.int32),
        ],
        mesh=mesh,
        compiler_params=pltpu.CompilerParams(needs_layout_passes=False),
        scratch_types=[
            pltpu.VMEM((L,), jnp.int32),
            pltpu.VMEM((T // CHUNK, CHUNK), jnp.int32),
            pltpu.VMEM((LANES,), jnp.int32),
            pltpu.VMEM((CHUNK, D), jnp.float32),
            pltpu.VMEM((CHUNK, D), jnp.float32),
            pltpu.VMEM((CHUNK, D), jnp.float32),
            pltpu.SemaphoreType.DMA,
            pltpu.SemaphoreType.DMA,
            pltpu.SemaphoreType.DMA,
            pltpu.SemaphoreType.DMA,
            pltpu.SemaphoreType.DMA,
            pltpu.SemaphoreType.DMA,
        ],
    )(xf, duration)
    mel_len = jnp.minimum(tot[:, 0], max_len).astype(jnp.int32)
    return out, mel_len


# speedup vs baseline: 1.0024x; 1.0024x over previous
import functools

import jax
import jax.numpy as jnp
from jax import lax
from jax.experimental import pallas as pl
from jax.experimental.pallas import tpu as pltpu
from jax.experimental.pallas import tpu_sc as plsc

B, L, D = 16, 512, 512
T = 2048
LANES = 16
NC, NS = 2, 16
NW = NC * NS
CHUNK = 32
NBUF = 6
HALF = T // 2
NCH = HALF // CHUNK
SHIFT = CHUNK.bit_length() - 1
PAD_ROWS = 8
SENTINEL = B * L


def _lr_body(xf, dur, out, tot, dur_v, idx_buf, tot_v,
             b0, b1, b2, b3, b4, b5, g0, g1, g2, g3, g4, g5,
             o0, o1, o2, o3, o4, o5):
    bufs = (b0, b1, b2, b3, b4, b5)
    gsem = (g0, g1, g2, g3, g4, g5)
    osem = (o0, o1, o2, o3, o4, o5)

    wid = lax.axis_index("c") * NS + lax.axis_index("s")
    b = wid // 2
    h = wid % 2

    pltpu.sync_copy(dur.at[b], dur_v)

    sent = jnp.full((LANES,), SENTINEL, jnp.int32)
    for r in range(T // CHUNK):
        for j in range(CHUNK // LANES):
            idx_buf[r, pl.ds(j * LANES, LANES)] = sent

    iota = lax.iota(jnp.int32, LANES)
    row_base = b * L

    def cs_body(i, carry):
        ch = dur_v[pl.ds(i * LANES, LANES)]
        inc = plsc.cumsum(ch)
        excl = carry + inc - ch
        src = row_base + i * LANES + iota
        for k in range(7):
            pos = excl + k
            m = (ch > k) & (pos < T)
            posc = jnp.minimum(pos, T - 1)
            plsc.store_scatter(idx_buf, [posc >> SHIFT, posc & (CHUNK - 1)],
                               src, mask=m)
        return carry + jnp.sum(ch)

    total = lax.fori_loop(0, L // LANES, cs_body, jnp.int32(0))

    @pl.when(h == 0)
    def _():
        tot_v[...] = jnp.full((LANES,), total, jnp.int32)
        pltpu.sync_copy(tot_v, tot.at[b])

    row0 = h * NCH

    def g_start(c, buf, sem):
        return pltpu.async_copy(xf.at[idx_buf.at[row0 + c]], buf, sem)

    def o_start(c, buf, sem):
        dst = out.at[b, pl.ds(h * HALF + c * CHUNK, CHUNK)]
        return pltpu.async_copy(buf, dst, sem)

    gh = {}
    oh = {}
    for c in range(min(NBUF, NCH)):
        gh[c] = g_start(c, bufs[c % NBUF], gsem[c % NBUF])
    for c in range(NCH):
        i = c % NBUF
        gh[c].wait()
        oh[c] = o_start(c, bufs[i], osem[i])
        p = c - 1
        if p >= 0 and p + NBUF < NCH:
            oh[p].wait()
            gh[p + NBUF] = g_start(p + NBUF, bufs[p % NBUF], gsem[p % NBUF])
    for c in range(max(0, NCH - NBUF), NCH):
        oh[c].wait()


def kernel(x, duration, max_len):
    xf = jnp.concatenate(
        [x.reshape(B * L, D), jnp.zeros((PAD_ROWS, D), x.dtype)], axis=0)
    mesh = plsc.VectorSubcoreMesh(core_axis_name="c", subcore_axis_name="s")
    out, tot = pl.kernel(
        _lr_body,
        out_type=[
            jax.ShapeDtypeStruct((B, T, D), x.dtype),
            jax.ShapeDtypeStruct((B, LANES), jnp.int32),
        ],
        mesh=mesh,
        compiler_params=pltpu.CompilerParams(needs_layout_passes=False),
        scratch_types=[
            pltpu.VMEM((L,), jnp.int32),
            pltpu.VMEM((T // CHUNK, CHUNK), jnp.int32),
            pltpu.VMEM((LANES,), jnp.int32),
            pltpu.VMEM((CHUNK, D), jnp.float32),
            pltpu.VMEM((CHUNK, D), jnp.float32),
            pltpu.VMEM((CHUNK, D), jnp.float32),
            pltpu.VMEM((CHUNK, D), jnp.float32),
            pltpu.VMEM((CHUNK, D), jnp.float32),
            pltpu.VMEM((CHUNK, D), jnp.float32),
            pltpu.SemaphoreType.DMA,
            pltpu.SemaphoreType.DMA,
            pltpu.SemaphoreType.DMA,
            pltpu.SemaphoreType.DMA,
            pltpu.SemaphoreType.DMA,
            pltpu.SemaphoreType.DMA,
            pltpu.SemaphoreType.DMA,
            pltpu.SemaphoreType.DMA,
            pltpu.SemaphoreType.DMA,
            pltpu.SemaphoreType.DMA,
            pltpu.SemaphoreType.DMA,
            pltpu.SemaphoreType.DMA,
        ],
    )(xf, duration)
    mel_len = jnp.minimum(tot[:, 0], max_len).astype(jnp.int32)
    return out, mel_len


# speedup vs baseline: 1.0998x; 1.0971x over previous
import jax
import jax.numpy as jnp
from jax import lax
from jax.experimental import pallas as pl
from jax.experimental.pallas import tpu as pltpu
from jax.experimental.pallas import tpu_sc as plsc

B, L, D = 16, 512, 512
T = 2048
LANES = 16
NC, NS = 2, 16
SCH = 32
NSC = L // SCH
BLK = 32
NBLK = T // BLK
SEG = D // LANES
UNROLL = 8
OBLK = 4


def _copy_row(dst_ref, dst_row, src_ref, src_row):
    def seg_body(sg, _):
        for jj in range(UNROLL):
            col = sg * (UNROLL * LANES) + jj * LANES
            dst_ref[dst_row, pl.ds(col, LANES)] = (
                src_ref[src_row, pl.ds(col, LANES)])
        return 0

    lax.fori_loop(0, SEG // UNROLL, seg_body, 0)


def _zero_row(dst_ref, dst_row):
    zeros = jnp.zeros((LANES,), jnp.float32)

    def seg_body(sg, _):
        for jj in range(UNROLL):
            col = sg * (UNROLL * LANES) + jj * LANES
            dst_ref[dst_row, pl.ds(col, LANES)] = zeros
        return 0

    lax.fori_loop(0, SEG // UNROLL, seg_body, 0)


def _lr_body(x, dur, out, tot, dur_v, tot_v, sbufA, sbufB, obuf, zbuf,
             dur_s, semA, semB, fsem):
    wid = lax.axis_index("c") * NS + lax.axis_index("s")
    b = wid // 2
    h = wid % 2

    pltpu.sync_copy(dur.at[b], dur_v)

    def sum_body(i, carry):
        return carry + jnp.sum(dur_v[pl.ds(i * LANES, LANES)])

    total = lax.fori_loop(0, L // LANES, sum_body, jnp.int32(0))
    mint = jnp.minimum(total, T)

    @pl.when(h == 1)
    def _zero_tail():
        tot_v[...] = jnp.full((LANES,), total, jnp.int32)
        pltpu.sync_copy(tot_v, tot.at[b])

        def zrow(r, _):
            _zero_row(zbuf, r)
            return 0

        lax.fori_loop(0, BLK, zrow, 0)
        z0 = (mint + BLK - 1) // BLK

        def zblk(k, _):
            pltpu.sync_copy(zbuf, out.at[b, pl.ds(k * BLK, BLK)])
            return 0

        lax.fori_loop(z0, NBLK, zblk, 0)

    @pl.when(h == 0)
    def _expand():
        pltpu.async_copy(x.at[b, pl.ds(0, SCH)], sbufA, semA)
        pltpu.async_copy(x.at[b, pl.ds(SCH, SCH)], sbufB, semB)

        def drain_to(lo, hi):
            def dbody(i, _):
                blk = lo + i
                pltpu.make_async_copy(
                    obuf.at[pl.ds((blk % OBLK) * BLK, BLK)],
                    out.at[b, pl.ds(blk * BLK, BLK)], fsem).wait()
                return 0
            lax.fori_loop(0, hi - lo, dbody, 0)

        def flush_range(lo, hi):
            def fbody(i, _):
                blk = lo + i
                pltpu.async_copy(
                    obuf.at[pl.ds((blk % OBLK) * BLK, BLK)],
                    out.at[b, pl.ds(blk * BLK, BLK)], fsem)
                return 0
            lax.fori_loop(0, hi - lo, fbody, 0)

        for gg in range(L // LANES):
            dv = dur_v[pl.ds(gg * LANES, LANES)]
            for lane in range(LANES):
                dur_s[gg * LANES + lane] = dv[lane]

        def pair_body(p, carry):
            t, fl, dn = carry
            for half, (sbuf, ssem) in enumerate(((sbufA, semA),
                                                 (sbufB, semB))):
                c = p * 2 + half
                pltpu.make_async_copy(
                    x.at[b, pl.ds(c * SCH, SCH)], sbuf, ssem).wait()

                def row_body(r, carry2, sbuf=sbuf):
                    t, fl, dn = carry2
                    cond = (r % 8) == 0
                    hi = jnp.maximum(dn, fl - 1)

                    @pl.when(cond)
                    def _():
                        drain_to(dn, hi)

                    dn = jnp.where(cond, hi, dn)
                    d_l = dur_s[c * SCH + r]
                    dk = jnp.maximum(0, jnp.minimum(d_l, T - t))

                    def kbody(k, _, t=t):
                        _copy_row(obuf, (t + k) % (OBLK * BLK), sbuf, r)
                        return 0

                    lax.fori_loop(0, dk, kbody, 0)
                    t = t + dk
                    cond2 = (r % 8) == 7
                    fl_new = jnp.where(cond2, t // BLK, fl)

                    @pl.when(cond2)
                    def _():
                        flush_range(fl, t // BLK)

                    return t, fl_new, dn

                t, fl, dn = lax.fori_loop(0, SCH, row_body, (t, fl, dn))

                @pl.when(c + 2 < NSC)
                def _prefetch(c=c, sbuf=sbuf, ssem=ssem):
                    pltpu.async_copy(
                        x.at[b, pl.ds((c + 2) * SCH, SCH)], sbuf, ssem)
            return t, fl, dn

        t, fl, dn = lax.fori_loop(
            0, NSC // 2, pair_body,
            (jnp.int32(0), jnp.int32(0), jnp.int32(0)))
        drain_to(dn, fl)

        npad = (BLK - t % BLK) % BLK

        def pad_body(k, _):
            _zero_row(obuf, (t + k) % (OBLK * BLK))
            return 0

        lax.fori_loop(0, npad, pad_body, 0)

        @pl.when(npad > 0)
        def _final_flush():
            pltpu.sync_copy(obuf.at[pl.ds((fl % OBLK) * BLK, BLK)],
                            out.at[b, pl.ds(fl * BLK, BLK)])


def kernel(x, duration, max_len):
    mesh = plsc.VectorSubcoreMesh(core_axis_name="c", subcore_axis_name="s")
    out, tot = pl.kernel(
        _lr_body,
        out_type=[
            jax.ShapeDtypeStruct((B, T, D), x.dtype),
            jax.ShapeDtypeStruct((B, LANES), jnp.int32),
        ],
        mesh=mesh,
        compiler_params=pltpu.CompilerParams(needs_layout_passes=False),
        scratch_types=[
            pltpu.VMEM((L,), jnp.int32),
            pltpu.VMEM((LANES,), jnp.int32),
            pltpu.VMEM((SCH, D), jnp.float32),
            pltpu.VMEM((SCH, D), jnp.float32),
            pltpu.VMEM((OBLK * BLK, D), jnp.float32),
            pltpu.VMEM((BLK, D), jnp.float32),
            pltpu.SMEM((L,), jnp.int32),
            pltpu.SemaphoreType.DMA,
            pltpu.SemaphoreType.DMA,
            pltpu.SemaphoreType.DMA,
        ],
    )(x, duration)
    mel_len = jnp.minimum(tot[:, 0], max_len).astype(jnp.int32)
    return out, mel_len


# speedup vs baseline: 1.1001x; 1.0003x over previous
import jax
import jax.numpy as jnp
from jax import lax
from jax.experimental import pallas as pl
from jax.experimental.pallas import tpu as pltpu
from jax.experimental.pallas import tpu_sc as plsc

B, L, D = 16, 512, 512
T = 2048
LANES = 16
NC, NS = 2, 16
SCH = 32
NSC = L // SCH
BLK = 32
NBLK = T // BLK
SEG = D // LANES
UNROLL = 8
OBLK = 4


def _copy_row(dst_ref, dst_row, src_ref, src_row):
    for j in range(SEG):
        col = j * LANES
        dst_ref[dst_row, pl.ds(col, LANES)] = (
            src_ref[src_row, pl.ds(col, LANES)])


def _zero_row(dst_ref, dst_row):
    zeros = jnp.zeros((LANES,), jnp.float32)
    for j in range(SEG):
        dst_ref[dst_row, pl.ds(j * LANES, LANES)] = zeros


def _lr_body(x, dur, out, tot, dur_v, tot_v, sbufA, sbufB, obuf, zbuf,
             dur_s, semA, semB, fsem):
    wid = lax.axis_index("c") * NS + lax.axis_index("s")
    b = wid // 2
    h = wid % 2

    pltpu.sync_copy(dur.at[b], dur_v)

    def sum_body(i, carry):
        return carry + jnp.sum(dur_v[pl.ds(i * LANES, LANES)])

    total = lax.fori_loop(0, L // LANES, sum_body, jnp.int32(0))
    mint = jnp.minimum(total, T)

    @pl.when(h == 1)
    def _zero_tail():
        tot_v[...] = jnp.full((LANES,), total, jnp.int32)
        pltpu.sync_copy(tot_v, tot.at[b])

        def zrow(r, _):
            _zero_row(zbuf, r)
            return 0

        lax.fori_loop(0, BLK, zrow, 0)
        z0 = (mint + BLK - 1) // BLK

        def zblk(k, _):
            pltpu.sync_copy(zbuf, out.at[b, pl.ds(k * BLK, BLK)])
            return 0

        lax.fori_loop(z0, NBLK, zblk, 0)

    @pl.when(h == 0)
    def _expand():
        pltpu.async_copy(x.at[b, pl.ds(0, SCH)], sbufA, semA)
        pltpu.async_copy(x.at[b, pl.ds(SCH, SCH)], sbufB, semB)

        def drain_to(lo, hi):
            def dbody(i, _):
                blk = lo + i
                pltpu.make_async_copy(
                    obuf.at[pl.ds((blk % OBLK) * BLK, BLK)],
                    out.at[b, pl.ds(blk * BLK, BLK)], fsem).wait()
                return 0
            lax.fori_loop(0, hi - lo, dbody, 0)

        def flush_range(lo, hi):
            def fbody(i, _):
                blk = lo + i
                pltpu.async_copy(
                    obuf.at[pl.ds((blk % OBLK) * BLK, BLK)],
                    out.at[b, pl.ds(blk * BLK, BLK)], fsem)
                return 0
            lax.fori_loop(0, hi - lo, fbody, 0)

        for gg in range(L // LANES):
            dv = dur_v[pl.ds(gg * LANES, LANES)]
            for lane in range(LANES):
                dur_s[gg * LANES + lane] = dv[lane]

        def pair_body(p, carry):
            t, fl, dn = carry
            for half, (sbuf, ssem) in enumerate(((sbufA, semA),
                                                 (sbufB, semB))):
                c = p * 2 + half
                pltpu.make_async_copy(
                    x.at[b, pl.ds(c * SCH, SCH)], sbuf, ssem).wait()

                def row_body(r, carry2, sbuf=sbuf):
                    t, fl, dn = carry2
                    cond = (r % 8) == 0
                    hi = jnp.maximum(dn, fl - 1)

                    @pl.when(cond)
                    def _():
                        drain_to(dn, hi)

                    dn = jnp.where(cond, hi, dn)
                    d_l = dur_s[c * SCH + r]
                    dk = jnp.maximum(0, jnp.minimum(d_l, T - t))

                    def kbody(k, _, t=t):
                        _copy_row(obuf, (t + k) % (OBLK * BLK), sbuf, r)
                        return 0

                    lax.fori_loop(0, dk, kbody, 0)
                    t = t + dk
                    cond2 = (r % 8) == 7
                    fl_new = jnp.where(cond2, t // BLK, fl)

                    @pl.when(cond2)
                    def _():
                        flush_range(fl, t // BLK)

                    return t, fl_new, dn

                t, fl, dn = lax.fori_loop(0, SCH, row_body, (t, fl, dn))

                @pl.when(c + 2 < NSC)
                def _prefetch(c=c, sbuf=sbuf, ssem=ssem):
                    pltpu.async_copy(
                        x.at[b, pl.ds((c + 2) * SCH, SCH)], sbuf, ssem)
            return t, fl, dn

        t, fl, dn = lax.fori_loop(
            0, NSC // 2, pair_body,
            (jnp.int32(0), jnp.int32(0), jnp.int32(0)))
        drain_to(dn, fl)

        npad = (BLK - t % BLK) % BLK

        def pad_body(k, _):
            _zero_row(obuf, (t + k) % (OBLK * BLK))
            return 0

        lax.fori_loop(0, npad, pad_body, 0)

        @pl.when(npad > 0)
        def _final_flush():
            pltpu.sync_copy(obuf.at[pl.ds((fl % OBLK) * BLK, BLK)],
                            out.at[b, pl.ds(fl * BLK, BLK)])


def kernel(x, duration, max_len):
    mesh = plsc.VectorSubcoreMesh(core_axis_name="c", subcore_axis_name="s")
    out, tot = pl.kernel(
        _lr_body,
        out_type=[
            jax.ShapeDtypeStruct((B, T, D), x.dtype),
            jax.ShapeDtypeStruct((B, LANES), jnp.int32),
        ],
        mesh=mesh,
        compiler_params=pltpu.CompilerParams(needs_layout_passes=False),
        scratch_types=[
            pltpu.VMEM((L,), jnp.int32),
            pltpu.VMEM((LANES,), jnp.int32),
            pltpu.VMEM((SCH, D), jnp.float32),
            pltpu.VMEM((SCH, D), jnp.float32),
            pltpu.VMEM((OBLK * BLK, D), jnp.float32),
            pltpu.VMEM((BLK, D), jnp.float32),
            pltpu.SMEM((L,), jnp.int32),
            pltpu.SemaphoreType.DMA,
            pltpu.SemaphoreType.DMA,
            pltpu.SemaphoreType.DMA,
        ],
    )(x, duration)
    mel_len = jnp.minimum(tot[:, 0], max_len).astype(jnp.int32)
    return out, mel_len


# speedup vs baseline: 2.3975x; 2.1794x over previous
import jax
import jax.numpy as jnp
from jax import lax
from jax.experimental import pallas as pl
from jax.experimental.pallas import tpu as pltpu
from jax.experimental.pallas import tpu_sc as plsc

B, L, D = 16, 512, 512
T = 2048
LANES = 16
NC, NS = 2, 16
SCH = 32
NSC = L // SCH
BLK = 32
NBLK = T // BLK
SEG = D // LANES
UNROLL = 8
OBLK = 4


def _copy_row(dst_ref, dst_row, src_ref, src_row):
    vals = [src_ref[src_row, pl.ds(j * LANES, LANES)] for j in range(SEG)]
    for j, v in enumerate(vals):
        dst_ref[dst_row, pl.ds(j * LANES, LANES)] = v


def _zero_row(dst_ref, dst_row):
    zeros = jnp.zeros((LANES,), jnp.float32)
    for j in range(SEG):
        dst_ref[dst_row, pl.ds(j * LANES, LANES)] = zeros


def _lr_body(x, dur, out, tot, dur_v, tot_v, sbufA, sbufB, obuf, zbuf,
             dur_s, semA, semB, fsem):
    wid = lax.axis_index("c") * NS + lax.axis_index("s")
    b = wid // 2
    h = wid % 2

    pltpu.sync_copy(dur.at[b], dur_v)

    def sum_body(i, carry):
        return carry + jnp.sum(dur_v[pl.ds(i * LANES, LANES)])

    total = lax.fori_loop(0, L // LANES, sum_body, jnp.int32(0))
    mint = jnp.minimum(total, T)

    @pl.when(h == 1)
    def _zero_tail():
        tot_v[...] = jnp.full((LANES,), total, jnp.int32)
        pltpu.sync_copy(tot_v, tot.at[b])

        def zrow(r, _):
            _zero_row(zbuf, r)
            return 0

        lax.fori_loop(0, BLK, zrow, 0)
        z0 = (mint + BLK - 1) // BLK

        def zblk(k, _):
            pltpu.sync_copy(zbuf, out.at[b, pl.ds(k * BLK, BLK)])
            return 0

        lax.fori_loop(z0, NBLK, zblk, 0)

    @pl.when(h == 0)
    def _expand():
        pltpu.async_copy(x.at[b, pl.ds(0, SCH)], sbufA, semA)
        pltpu.async_copy(x.at[b, pl.ds(SCH, SCH)], sbufB, semB)

        def drain_to(lo, hi):
            def dbody(i, _):
                blk = lo + i
                pltpu.make_async_copy(
                    obuf.at[pl.ds((blk % OBLK) * BLK, BLK)],
                    out.at[b, pl.ds(blk * BLK, BLK)], fsem).wait()
                return 0
            lax.fori_loop(0, hi - lo, dbody, 0)

        def flush_range(lo, hi):
            def fbody(i, _):
                blk = lo + i
                pltpu.async_copy(
                    obuf.at[pl.ds((blk % OBLK) * BLK, BLK)],
                    out.at[b, pl.ds(blk * BLK, BLK)], fsem)
                return 0
            lax.fori_loop(0, hi - lo, fbody, 0)

        for gg in range(L // LANES):
            dv = dur_v[pl.ds(gg * LANES, LANES)]
            for lane in range(LANES):
                dur_s[gg * LANES + lane] = dv[lane]

        def pair_body(p, carry):
            t, fl, dn = carry
            for half, (sbuf, ssem) in enumerate(((sbufA, semA),
                                                 (sbufB, semB))):
                c = p * 2 + half
                pltpu.make_async_copy(
                    x.at[b, pl.ds(c * SCH, SCH)], sbuf, ssem).wait()

                def row_body(r, carry2, sbuf=sbuf):
                    t, fl, dn = carry2
                    cond = (r % 8) == 0
                    hi = jnp.maximum(dn, fl - 1)

                    @pl.when(cond)
                    def _():
                        drain_to(dn, hi)

                    dn = jnp.where(cond, hi, dn)
                    d_l = dur_s[c * SCH + r]
                    dk = jnp.maximum(0, jnp.minimum(d_l, T - t))

                    def kbody(k, _, t=t):
                        _copy_row(obuf, (t + k) % (OBLK * BLK), sbuf, r)
                        return 0

                    lax.fori_loop(0, dk, kbody, 0)
                    t = t + dk
                    cond2 = (r % 8) == 7
                    fl_new = jnp.where(cond2, t // BLK, fl)

                    @pl.when(cond2)
                    def _():
                        flush_range(fl, t // BLK)

                    return t, fl_new, dn

                t, fl, dn = lax.fori_loop(0, SCH, row_body, (t, fl, dn))

                @pl.when(c + 2 < NSC)
                def _prefetch(c=c, sbuf=sbuf, ssem=ssem):
                    pltpu.async_copy(
                        x.at[b, pl.ds((c + 2) * SCH, SCH)], sbuf, ssem)
            return t, fl, dn

        t, fl, dn = lax.fori_loop(
            0, NSC // 2, pair_body,
            (jnp.int32(0), jnp.int32(0), jnp.int32(0)))
        drain_to(dn, fl)

        npad = (BLK - t % BLK) % BLK

        def pad_body(k, _):
            _zero_row(obuf, (t + k) % (OBLK * BLK))
            return 0

        lax.fori_loop(0, npad, pad_body, 0)

        @pl.when(npad > 0)
        def _final_flush():
            pltpu.sync_copy(obuf.at[pl.ds((fl % OBLK) * BLK, BLK)],
                            out.at[b, pl.ds(fl * BLK, BLK)])


def kernel(x, duration, max_len):
    mesh = plsc.VectorSubcoreMesh(core_axis_name="c", subcore_axis_name="s")
    out, tot = pl.kernel(
        _lr_body,
        out_type=[
            jax.ShapeDtypeStruct((B, T, D), x.dtype),
            jax.ShapeDtypeStruct((B, LANES), jnp.int32),
        ],
        mesh=mesh,
        compiler_params=pltpu.CompilerParams(needs_layout_passes=False),
        scratch_types=[
            pltpu.VMEM((L,), jnp.int32),
            pltpu.VMEM((LANES,), jnp.int32),
            pltpu.VMEM((SCH, D), jnp.float32),
            pltpu.VMEM((SCH, D), jnp.float32),
            pltpu.VMEM((OBLK * BLK, D), jnp.float32),
            pltpu.VMEM((BLK, D), jnp.float32),
            pltpu.SMEM((L,), jnp.int32),
            pltpu.SemaphoreType.DMA,
            pltpu.SemaphoreType.DMA,
            pltpu.SemaphoreType.DMA,
        ],
    )(x, duration)
    mel_len = jnp.minimum(tot[:, 0], max_len).astype(jnp.int32)
    return out, mel_len


# speedup vs baseline: 3.1490x; 1.3134x over previous
import jax
import jax.numpy as jnp
from jax import lax
from jax.experimental import pallas as pl
from jax.experimental.pallas import tpu as pltpu
from jax.experimental.pallas import tpu_sc as plsc

B, L, D = 16, 512, 512
T = 2048
LANES = 16
NC, NS = 2, 16
SCH = 32
NSC = L // SCH
BLK = 32
NBLK = T // BLK
SEG = D // LANES
UNROLL = 8
OBLK = 4


def _copy_row(dst_ref, dst_row, src_ref, src_row):
    vals = [src_ref[src_row, pl.ds(j * LANES, LANES)] for j in range(SEG)]
    for j, v in enumerate(vals):
        dst_ref[dst_row, pl.ds(j * LANES, LANES)] = v


def _zero_row(dst_ref, dst_row):
    zeros = jnp.zeros((LANES,), jnp.float32)
    for j in range(SEG):
        dst_ref[dst_row, pl.ds(j * LANES, LANES)] = zeros


def _lr_body(x, dur, out, tot, dur_v, tot_v, sbufA, sbufB, obuf, zbuf,
             dur_s, semA, semB, fsem):
    wid = lax.axis_index("c") * NS + lax.axis_index("s")
    b = wid // 2
    h = wid % 2

    pltpu.sync_copy(dur.at[b], dur_v)

    def sum_body(i, carry):
        return carry + jnp.sum(dur_v[pl.ds(i * LANES, LANES)])

    t_mid = lax.fori_loop(0, L // (2 * LANES), sum_body, jnp.int32(0))
    total = lax.fori_loop(L // (2 * LANES), L // LANES, sum_body, t_mid)
    mint = jnp.minimum(total, T)

    stop0 = (jnp.minimum(t_mid, T) // BLK) * BLK
    w_lo = jnp.where(h == 0, 0, stop0)
    w_hi = jnp.where(h == 0, stop0, T)

    @pl.when(h == 1)
    def _tot_write():
        tot_v[...] = jnp.full((LANES,), total, jnp.int32)
        pltpu.sync_copy(tot_v, tot.at[b])

    for gg in range(L // LANES):
        dv = dur_v[pl.ds(gg * LANES, LANES)]
        for lane in range(LANES):
            dur_s[gg * LANES + lane] = dv[lane]

    pltpu.async_copy(x.at[b, pl.ds(0, SCH)], sbufA, semA)
    pltpu.async_copy(x.at[b, pl.ds(SCH, SCH)], sbufB, semB)

    def drain_to(lo, hi):
        def dbody(i, _):
            blk = lo + i
            pltpu.make_async_copy(
                obuf.at[pl.ds((blk % OBLK) * BLK, BLK)],
                out.at[b, pl.ds(blk * BLK, BLK)], fsem).wait()
            return 0
        lax.fori_loop(0, hi - lo, dbody, 0)

    def flush_range(lo, hi):
        def fbody(i, _):
            blk = lo + i
            pltpu.async_copy(
                obuf.at[pl.ds((blk % OBLK) * BLK, BLK)],
                out.at[b, pl.ds(blk * BLK, BLK)], fsem)
            return 0
        lax.fori_loop(0, hi - lo, fbody, 0)

    fl0 = w_lo // BLK
    cap = w_hi // BLK

    def pair_body(p, carry):
        t, fl, dn = carry
        for half, (sbuf, ssem) in enumerate(((sbufA, semA),
                                             (sbufB, semB))):
            c = p * 2 + half
            pltpu.make_async_copy(
                x.at[b, pl.ds(c * SCH, SCH)], sbuf, ssem).wait()

            def row_body(r, carry2, sbuf=sbuf):
                t, fl, dn = carry2
                cond = (r % 8) == 0
                hi = jnp.maximum(dn, fl - 1)

                @pl.when(cond)
                def _():
                    drain_to(dn, hi)

                dn = jnp.where(cond, hi, dn)
                d_l = dur_s[c * SCH + r]
                dk = jnp.maximum(0, jnp.minimum(d_l, T - t))
                k_lo = jnp.clip(w_lo - t, 0, dk)
                k_hi = jnp.clip(w_hi - t, 0, dk)

                def kbody(k, _, t=t):
                    _copy_row(obuf, (t + k) % (OBLK * BLK), sbuf, r)
                    return 0

                lax.fori_loop(k_lo, k_hi, kbody, 0)
                t = t + dk
                cond2 = (r % 8) == 7
                fl_new = jnp.where(cond2,
                                   jnp.clip(t // BLK, fl, cap), fl)

                @pl.when(cond2)
                def _():
                    flush_range(fl, fl_new)

                return t, fl_new, dn

            t, fl, dn = lax.fori_loop(0, SCH, row_body, (t, fl, dn))

            @pl.when(c + 2 < NSC)
            def _prefetch(c=c, sbuf=sbuf, ssem=ssem):
                pltpu.async_copy(
                    x.at[b, pl.ds((c + 2) * SCH, SCH)], sbuf, ssem)
        return t, fl, dn

    t, fl, dn = lax.fori_loop(0, NSC // 2, pair_body,
                              (jnp.int32(0), fl0, fl0))
    drain_to(dn, fl)

    @pl.when(h == 1)
    def _pad_and_tail():
        npad = (BLK - mint % BLK) % BLK

        def pad_body(k, _):
            _zero_row(obuf, (mint + k) % (OBLK * BLK))
            return 0

        lax.fori_loop(0, npad, pad_body, 0)

        @pl.when(npad > 0)
        def _final_flush():
            pltpu.sync_copy(obuf.at[pl.ds(((mint // BLK) % OBLK) * BLK, BLK)],
                            out.at[b, pl.ds((mint // BLK) * BLK, BLK)])

        def zrow(r, _):
            _zero_row(zbuf, r)
            return 0

        lax.fori_loop(0, BLK, zrow, 0)
        z0 = (mint + BLK - 1) // BLK

        def zblk(k, _):
            pltpu.sync_copy(zbuf, out.at[b, pl.ds(k * BLK, BLK)])
            return 0

        lax.fori_loop(z0, NBLK, zblk, 0)


def kernel(x, duration, max_len):
    mesh = plsc.VectorSubcoreMesh(core_axis_name="c", subcore_axis_name="s")
    out, tot = pl.kernel(
        _lr_body,
        out_type=[
            jax.ShapeDtypeStruct((B, T, D), x.dtype),
            jax.ShapeDtypeStruct((B, LANES), jnp.int32),
        ],
        mesh=mesh,
        compiler_params=pltpu.CompilerParams(needs_layout_passes=False),
        scratch_types=[
            pltpu.VMEM((L,), jnp.int32),
            pltpu.VMEM((LANES,), jnp.int32),
            pltpu.VMEM((SCH, D), jnp.float32),
            pltpu.VMEM((SCH, D), jnp.float32),
            pltpu.VMEM((OBLK * BLK, D), jnp.float32),
            pltpu.VMEM((BLK, D), jnp.float32),
            pltpu.SMEM((L,), jnp.int32),
            pltpu.SemaphoreType.DMA,
            pltpu.SemaphoreType.DMA,
            pltpu.SemaphoreType.DMA,
        ],
    )(x, duration)
    mel_len = jnp.minimum(tot[:, 0], max_len).astype(jnp.int32)
    return out, mel_len


# speedup vs baseline: 3.1671x; 1.0058x over previous
import jax
import jax.numpy as jnp
from jax import lax
from jax.experimental import pallas as pl
from jax.experimental.pallas import tpu as pltpu
from jax.experimental.pallas import tpu_sc as plsc

B, L, D = 16, 512, 512
T = 2048
LANES = 16
NC, NS = 2, 16
SCH = 32
NSC = L // SCH
BLK = 32
NBLK = T // BLK
SEG = D // LANES
UNROLL = 8
OBLK = 4


def _copy_row(dst_ref, dst_row, src_ref, src_row):
    vals = [src_ref[src_row, pl.ds(j * LANES, LANES)] for j in range(SEG)]
    for j, v in enumerate(vals):
        dst_ref[dst_row, pl.ds(j * LANES, LANES)] = v


def _zero_row(dst_ref, dst_row):
    zeros = jnp.zeros((LANES,), jnp.float32)
    for j in range(SEG):
        dst_ref[dst_row, pl.ds(j * LANES, LANES)] = zeros


def _lr_body(x, dur, out, tot, dur_v, tot_v, sbufA, sbufB, obuf, zbuf,
             dur_s, semA, semB, fsem):
    wid = lax.axis_index("c") * NS + lax.axis_index("s")
    b = wid // 2
    h = wid % 2

    pltpu.sync_copy(dur.at[b], dur_v)

    def sum_body(i, carry):
        return carry + jnp.sum(dur_v[pl.ds(i * LANES, LANES)])

    t_mid = lax.fori_loop(0, L // (2 * LANES), sum_body, jnp.int32(0))
    total = lax.fori_loop(L // (2 * LANES), L // LANES, sum_body, t_mid)
    mint = jnp.minimum(total, T)

    stop0 = (jnp.minimum(t_mid, T) // BLK) * BLK
    w_lo = jnp.where(h == 0, 0, stop0)
    w_hi = jnp.where(h == 0, stop0, T)

    @pl.when(h == 1)
    def _tot_write():
        tot_v[...] = jnp.full((LANES,), total, jnp.int32)
        pltpu.sync_copy(tot_v, tot.at[b])

    for gg in range(L // LANES):
        dv = dur_v[pl.ds(gg * LANES, LANES)]
        for lane in range(LANES):
            dur_s[gg * LANES + lane] = dv[lane]

    pltpu.async_copy(x.at[b, pl.ds(0, SCH)], sbufA, semA)
    pltpu.async_copy(x.at[b, pl.ds(SCH, SCH)], sbufB, semB)

    def drain_to(lo, hi):
        def dbody(i, _):
            blk = lo + i
            pltpu.make_async_copy(
                obuf.at[pl.ds((blk % OBLK) * BLK, BLK)],
                out.at[b, pl.ds(blk * BLK, BLK)], fsem).wait()
            return 0
        lax.fori_loop(0, hi - lo, dbody, 0)

    def flush_range(lo, hi):
        def fbody(i, _):
            blk = lo + i
            pltpu.async_copy(
                obuf.at[pl.ds((blk % OBLK) * BLK, BLK)],
                out.at[b, pl.ds(blk * BLK, BLK)], fsem)
            return 0
        lax.fori_loop(0, hi - lo, fbody, 0)

    fl0 = w_lo // BLK
    cap = w_hi // BLK

    def pair_body(p, carry):
        t, fl, dn = carry
        for half, (sbuf, ssem) in enumerate(((sbufA, semA),
                                             (sbufB, semB))):
            c = p * 2 + half
            pltpu.make_async_copy(
                x.at[b, pl.ds(c * SCH, SCH)], sbuf, ssem).wait()

            def row_body(r, carry2, sbuf=sbuf):
                t, fl, dn = carry2
                cond = (r % 8) == 0
                hi = jnp.maximum(dn, fl - 1)

                @pl.when(cond)
                def _():
                    drain_to(dn, hi)

                dn = jnp.where(cond, hi, dn)
                d_l = dur_s[c * SCH + r]
                dk = jnp.maximum(0, jnp.minimum(d_l, T - t))
                k_lo = jnp.clip(w_lo - t, 0, dk)
                k_hi = jnp.clip(w_hi - t, 0, dk)

                def kbody(k, _, t=t):
                    _copy_row(obuf, (t + k) % (OBLK * BLK), sbuf, r)
                    return 0

                lax.fori_loop(k_lo, k_hi, kbody, 0)
                t = t + dk
                cond2 = (r % 8) == 7
                fl_new = jnp.where(cond2,
                                   jnp.clip(t // BLK, fl, cap), fl)

                @pl.when(cond2)
                def _():
                    flush_range(fl, fl_new)

                return t, fl_new, dn

            t, fl, dn = lax.fori_loop(0, SCH, row_body, (t, fl, dn))

            @pl.when(c + 2 < NSC)
            def _prefetch(c=c, sbuf=sbuf, ssem=ssem):
                pltpu.async_copy(
                    x.at[b, pl.ds((c + 2) * SCH, SCH)], sbuf, ssem)
        return t, fl, dn

    t, fl, dn = lax.fori_loop(0, NSC // 2, pair_body,
                              (jnp.int32(0), fl0, fl0))
    drain_to(dn, fl)

    @pl.when(h == 1)
    def _pad_and_tail():
        npad = (BLK - mint % BLK) % BLK

        def pad_body(k, _):
            _zero_row(obuf, (mint + k) % (OBLK * BLK))
            return 0

        lax.fori_loop(0, npad, pad_body, 0)

        @pl.when(npad > 0)
        def _final_flush():
            pltpu.sync_copy(obuf.at[pl.ds(((mint // BLK) % OBLK) * BLK, BLK)],
                            out.at[b, pl.ds((mint // BLK) * BLK, BLK)])

        def zrow(r, _):
            _zero_row(zbuf, r)
            return 0

        lax.fori_loop(0, BLK, zrow, 0)
        z0 = (mint + BLK - 1) // BLK

        def zblk(k, _):
            pltpu.async_copy(zbuf, out.at[b, pl.ds(k * BLK, BLK)], fsem)
            return 0

        lax.fori_loop(z0, NBLK, zblk, 0)

        def zdrain(k, _):
            pltpu.make_async_copy(
                zbuf, out.at[b, pl.ds(k * BLK, BLK)], fsem).wait()
            return 0

        lax.fori_loop(z0, NBLK, zdrain, 0)


def kernel(x, duration, max_len):
    mesh = plsc.VectorSubcoreMesh(core_axis_name="c", subcore_axis_name="s")
    out, tot = pl.kernel(
        _lr_body,
        out_type=[
            jax.ShapeDtypeStruct((B, T, D), x.dtype),
            jax.ShapeDtypeStruct((B, LANES), jnp.int32),
        ],
        mesh=mesh,
        compiler_params=pltpu.CompilerParams(needs_layout_passes=False),
        scratch_types=[
            pltpu.VMEM((L,), jnp.int32),
            pltpu.VMEM((LANES,), jnp.int32),
            pltpu.VMEM((SCH, D), jnp.float32),
            pltpu.VMEM((SCH, D), jnp.float32),
            pltpu.VMEM((OBLK * BLK, D), jnp.float32),
            pltpu.VMEM((BLK, D), jnp.float32),
            pltpu.SMEM((L,), jnp.int32),
            pltpu.SemaphoreType.DMA,
            pltpu.SemaphoreType.DMA,
            pltpu.SemaphoreType.DMA,
        ],
    )(x, duration)
    mel_len = jnp.minimum(tot[:, 0], max_len).astype(jnp.int32)
    return out, mel_len
